# baseline (speedup 1.0000x reference)
import functools

import jax
import jax.numpy as jnp
from jax.experimental import pallas as pl

N_USERS = 30000
N_ITEMS = 20000
N_NODES = 100000
EMB = 32
NEG_SLOPE = 0.01

_BLK = 2000


def _linear_leaky_kernel(x_ref, w_ref, b_ref, o_ref):
    y = jnp.dot(x_ref[...], w_ref[...], preferred_element_type=jnp.float32)
    y = y + b_ref[...]
    o_ref[...] = jnp.where(y >= 0, y, NEG_SLOPE * y)


def _linear_leaky(x, W, b):
    return pl.pallas_call(
        _linear_leaky_kernel,
        grid=(N_NODES // _BLK,),
        in_specs=[
            pl.BlockSpec((_BLK, EMB), lambda i: (i, 0)),
            pl.BlockSpec((EMB, EMB), lambda i: (0, 0)),
            pl.BlockSpec((1, EMB), lambda i: (0, 0)),
        ],
        out_specs=pl.BlockSpec((_BLK, EMB), lambda i: (i, 0)),
        out_shape=jax.ShapeDtypeStruct((N_NODES, EMB), jnp.float32),
    )(x, W, b.reshape(1, EMB))


def kernel(embedding, edge_index, edge_weight, W1, b1, W2, b2):
    row = edge_index[0]
    col = edge_index[1]

    def spmm(x):
        gathered = x[col] * edge_weight[:, None]
        return jax.ops.segment_sum(gathered, row, num_segments=N_NODES)

    acc = embedding
    ego = embedding
    for (W, b) in ((W1, b1), (W2, b2)):
        side = spmm(ego)
        ego = _linear_leaky(side, W, b)
        acc = acc + ego
    out = acc * (1.0 / 3.0)
    return (out[:N_USERS], out[N_USERS:N_USERS + N_ITEMS])


# speedup vs baseline: 15.2671x; 15.2671x over previous
import functools

import jax
import jax.numpy as jnp
from jax import lax
from jax.experimental import pallas as pl
from jax.experimental.pallas import tpu as pltpu
from jax.experimental.pallas import tpu_sc as plsc

N_USERS = 30000
N_ITEMS = 20000
N_NODES = 100000
N_HALF = 50000
EMB = 32
NEG_SLOPE = 0.01
E = 1600000

HP = 50048
NP = 2 * HP
PAD = 48
ROWS_PER_TILE = HP // 16

EPT = E // 32
CHUNK_P = 400
NK_P = EPT // CHUNK_P
GROUPS_P = CHUNK_P // 16
REGION_BLKS = 394
TOTBLK = 64 * REGION_BLKS

CHUNK = 256
SUB = 128
NSUB = CHUNK // SUB

_mesh = plsc.VectorSubcoreMesh(
    core_axis_name="c", subcore_axis_name="s", num_cores=2, num_subcores=16)

_params = pltpu.CompilerParams(
    needs_layout_passes=False, use_tc_tiling_on_sc=False)


def _splat(wvec, l):
    return lax.gather(
        wvec, jnp.full((16, 1), l, jnp.int32),
        lax.GatherDimensionNumbers(
            offset_dims=(), collapsed_slice_dims=(0,), start_index_map=(0,)),
        (1,), mode=lax.GatherScatterMode.PROMISE_IN_BOUNDS)


def _part_body(row_hbm, col_hbm, w_hbm,
               prow_hbm, pcol_hbm, pw_hbm, cnts_hbm,
               srow0, scol0, sw0, srow1, scol1, sw1,
               abi0, abc0, abw0, abi1, abc1, abw1,
               cbuf, ssem0, ssem1, fsem0, fsem1):
    c = lax.axis_index("c")
    s = lax.axis_index("s")
    t = c * 16 + s
    ebase0 = t * EPT

    stage_bufs = ((srow0, scol0, sw0, ssem0), (srow1, scol1, sw1, ssem1))
    ap_bufs = ((abi0, abc0, abw0, fsem0), (abi1, abc1, abw1, fsem1))
    rbase = (t * 2, t * 2 + 1)

    def stage(k, buf):
        srow, scol, sw, ssem = buf
        eb = ebase0 + k * CHUNK_P
        pltpu.async_copy(row_hbm.at[pl.ds(eb, CHUNK_P)], srow, ssem)
        pltpu.async_copy(col_hbm.at[pl.ds(eb, CHUNK_P)], scol, ssem)
        pltpu.async_copy(w_hbm.at[pl.ds(eb, CHUNK_P)], sw, ssem)

    def wait_stage(buf):
        srow, scol, sw, ssem = buf
        pltpu.make_async_copy(row_hbm.at[pl.ds(0, CHUNK_P)], srow, ssem).wait()
        pltpu.make_async_copy(col_hbm.at[pl.ds(0, CHUNK_P)], scol, ssem).wait()
        pltpu.make_async_copy(w_hbm.at[pl.ds(0, CHUNK_P)], sw, ssem).wait()

    def flush(h, p, bidx):
        abi, abc, abw, fsem = ap_bufs[h]
        o = 128 * p
        pltpu.async_copy(abi.at[pl.ds(0, 1), pl.ds(o, 128)],
                         prow_hbm.at[pl.ds(bidx, 1)], fsem)
        pltpu.async_copy(abc.at[pl.ds(0, 1), pl.ds(o, 128)],
                         pcol_hbm.at[pl.ds(bidx, 1)], fsem)
        pltpu.async_copy(abw.at[pl.ds(0, 1), pl.ds(o, 128)],
                         pw_hbm.at[pl.ds(bidx, 1)], fsem)

    def wait_flush(h):
        abi, abc, abw, fsem = ap_bufs[h]
        pltpu.make_async_copy(abi.at[pl.ds(0, 1), pl.ds(0, 128)],
                              prow_hbm.at[pl.ds(0, 1)], fsem).wait()
        pltpu.make_async_copy(abc.at[pl.ds(0, 1), pl.ds(0, 128)],
                              pcol_hbm.at[pl.ds(0, 1)], fsem).wait()
        pltpu.make_async_copy(abw.at[pl.ds(0, 1), pl.ds(0, 128)],
                              pw_hbm.at[pl.ds(0, 1)], fsem).wait()

    z16i = jnp.zeros((16,), jnp.int32)
    z16f = jnp.zeros((16,), jnp.float32)

    def append(h, vals, mask, off, blk):
        abi, abc, abw, _ = ap_bufs[h]
        rl, cp, wv = vals
        plsc.store_compressed(abi.at[0, pl.ds(off, 16)], rl, mask=mask)
        plsc.store_compressed(abc.at[0, pl.ds(off, 16)], cp, mask=mask)
        plsc.store_compressed(abw.at[0, pl.ds(off, 16)], wv, mask=mask)
        return off, blk

    def do_flush_check(h, off, blk):
        p = blk & 1
        thr = 128 + 128 * p
        cond = off >= thr

        @pl.when(cond)
        def _():
            @pl.when(blk >= 1)
            def _():
                wait_flush(h)
            flush(h, p, rbase[h] * REGION_BLKS + blk)

            @pl.when(p == 1)
            def _():
                abi, abc, abw, _ = ap_bufs[h]
                abi[0, pl.ds(0, 16)] = abi[0, pl.ds(256, 16)]
                abc[0, pl.ds(0, 16)] = abc[0, pl.ds(256, 16)]
                abw[0, pl.ds(0, 16)] = abw[0, pl.ds(256, 16)]

        off = jnp.where(cond & (p == 1), off - 256, off)
        blk = jnp.where(cond, blk + 1, blk)
        return off, blk

    def step(k, pbuf, carry):
        @pl.when(k + 1 < NK_P)
        def _():
            stage(k + 1, stage_bufs[1 - pbuf])

        wait_stage(stage_bufs[pbuf])
        srow, scol, sw, _ = stage_bufs[pbuf]

        def grp(g, carry):
            off0, blk0, off1, blk1 = carry
            e0 = g * 16
            r = srow[pl.ds(e0, 16)]
            cv = scol[pl.ds(e0, 16)]
            wv = sw[pl.ds(e0, 16)]
            m1 = r >= N_HALF
            rl = r - jnp.where(m1, N_HALF, 0)
            cp = cv + jnp.where(cv >= N_HALF, PAD, 0)
            append(0, (rl, cp, wv), ~m1, off0, blk0)
            append(1, (rl, cp, wv), m1, off1, blk1)
            cnt1 = jnp.sum(m1.astype(jnp.int32))
            off0 = off0 + (16 - cnt1)
            off1 = off1 + cnt1
            off0, blk0 = do_flush_check(0, off0, blk0)
            off1, blk1 = do_flush_check(1, off1, blk1)
            return off0, blk0, off1, blk1

        return lax.fori_loop(0, GROUPS_P, grp, carry)

    stage(0, stage_bufs[0])

    def pair(m, carry):
        carry = step(2 * m, 0, carry)
        carry = step(2 * m + 1, 1, carry)
        return carry

    carry = lax.fori_loop(0, NK_P // 2, pair,
                          (jnp.int32(0), jnp.int32(0),
                           jnp.int32(0), jnp.int32(0)))
    carry = step(NK_P - 1, 0, carry)
    off0, blk0, off1, blk1 = carry

    def finalize(h, off, blk):
        abi, abc, abw, _ = ap_bufs[h]
        p = blk & 1
        for j in range(8):
            pos = off + 16 * j

            @pl.when(pos < 128 * (p + 1))
            def _():
                abi[0, pl.ds(pos, 16)] = z16i
                abc[0, pl.ds(pos, 16)] = z16i
                abw[0, pl.ds(pos, 16)] = z16f

        @pl.when(blk >= 1)
        def _():
            wait_flush(h)

        flush(h, p, rbase[h] * REGION_BLKS + blk)
        wait_flush(h)
        blk = blk + 1
        q = blk & 1

        @pl.when(q == 1)
        def _():
            for j in range(8):
                abi[0, pl.ds(128 + 16 * j, 16)] = z16i
                abc[0, pl.ds(128 + 16 * j, 16)] = z16i
                abw[0, pl.ds(128 + 16 * j, 16)] = z16f
            flush(h, 1, rbase[h] * REGION_BLKS + blk)
            wait_flush(h)

        blk = jnp.where(q == 1, blk + 1, blk)
        cbuf[pl.ds(0, 16)] = z16i + blk
        pltpu.sync_copy(cbuf, cnts_hbm.at[pl.ds(rbase[h] * 16, 16)])

    finalize(0, off0, blk0)
    finalize(1, off1, blk1)


_part_call = pl.kernel(
    _part_body,
    out_type=(
        jax.ShapeDtypeStruct((TOTBLK, SUB), jnp.int32),
        jax.ShapeDtypeStruct((TOTBLK, SUB), jnp.int32),
        jax.ShapeDtypeStruct((TOTBLK, SUB), jnp.float32),
        jax.ShapeDtypeStruct((64 * 16,), jnp.int32),
    ),
    mesh=_mesh,
    scratch_types=(
        [pltpu.VMEM((CHUNK_P,), jnp.int32),
         pltpu.VMEM((CHUNK_P,), jnp.int32),
         pltpu.VMEM((CHUNK_P,), jnp.float32)] * 2
        + [pltpu.VMEM((1, 272), jnp.int32),
           pltpu.VMEM((1, 272), jnp.int32),
           pltpu.VMEM((1, 272), jnp.float32)] * 2
        + [pltpu.VMEM((16,), jnp.int32)]
        + [pltpu.SemaphoreType.DMA] * 4
    ),
    compiler_params=_params,
)


def _spmm_body(x_hbm, prow_hbm, pcol_hbm, pw_hbm, cnts_hbm, z_hbm, out_hbm,
               idx_a, col_a, w_a, rows_a,
               idx_b, col_b, w_b, rows_b,
               idx_c, col_c, w_c, rows_c,
               cntv, acc,
               ssem_a, ssem_b, ssem_c, gsem_a, gsem_b, gsem_c,
               csem_a, csem_b, csem_c):
    c = lax.axis_index("c")
    s = lax.axis_index("s")

    bufs = ((idx_a, col_a, w_a, rows_a, ssem_a, gsem_a, csem_a),
            (idx_b, col_b, w_b, rows_b, ssem_b, gsem_b, csem_b),
            (idx_c, col_c, w_c, rows_c, ssem_c, gsem_c, csem_c))

    pltpu.sync_copy(z_hbm, acc.at[pl.ds(s * ROWS_PER_TILE, ROWS_PER_TILE)])
    plsc.subcore_barrier()

    pltpu.sync_copy(cnts_hbm, cntv)
    rA = 4 * s + c
    rB = 4 * s + 2 + c
    nkA = cntv[pl.ds(rA * 16, 16)][0] // 2
    nkB = cntv[pl.ds(rB * 16, 16)][0] // 2
    nk = nkA + nkB
    baseA = rA * REGION_BLKS
    baseB = rB * REGION_BLKS

    def bidx_of(k):
        return jnp.where(k < nkA, baseA + 2 * k, baseB + 2 * (k - nkA))

    def stage(k, buf):
        idx_v, col_v, w_v, _, ssem, _, _ = buf
        b = bidx_of(k)
        pltpu.async_copy(prow_hbm.at[pl.ds(b, NSUB)], idx_v, ssem)
        pltpu.async_copy(pcol_hbm.at[pl.ds(b, NSUB)], col_v, ssem)
        pltpu.async_copy(pw_hbm.at[pl.ds(b, NSUB)], w_v, ssem)

    def wait_stage(buf):
        idx_v, col_v, w_v, _, ssem, _, _ = buf
        pltpu.make_async_copy(prow_hbm.at[pl.ds(0, NSUB)], idx_v, ssem).wait()
        pltpu.make_async_copy(pcol_hbm.at[pl.ds(0, NSUB)], col_v, ssem).wait()
        pltpu.make_async_copy(pw_hbm.at[pl.ds(0, NSUB)], w_v, ssem).wait()

    def gather(buf):
        _, col_v, _, rows_v, _, gsem, _ = buf
        for j in range(NSUB):
            pltpu.async_copy(x_hbm.at[col_v.at[j]],
                             rows_v.at[pl.ds(j * SUB, SUB)], gsem)

    def wait_gather(buf):
        _, col_v, _, rows_v, _, gsem, _ = buf
        for j in range(NSUB):
            pltpu.make_async_copy(x_hbm.at[col_v.at[j]],
                                  rows_v.at[pl.ds(j * SUB, SUB)], gsem).wait()

    def compute(buf):
        _, _, w_v, rows_v, _, _, _ = buf

        def grp(g, _):
            wvec = w_v[g >> 3, pl.ds((g & 7) * 16, 16)]
            e0 = g * 16
            for l in range(16):
                spl = _splat(wvec, l)
                rows_v[e0 + l, pl.ds(0, 16)] = \
                    rows_v[e0 + l, pl.ds(0, 16)] * spl
                rows_v[e0 + l, pl.ds(16, 16)] = \
                    rows_v[e0 + l, pl.ds(16, 16)] * spl
            return 0

        lax.fori_loop(0, CHUNK // 16, grp, 0)

    def scatter(buf):
        idx_v, _, _, rows_v, _, _, csem = buf
        for j in range(NSUB):
            pltpu.async_copy(rows_v.at[pl.ds(j * SUB, SUB)],
                             acc.at[idx_v.at[j]], csem, add=True)

    def wait_scatter(buf):
        idx_v, _, _, rows_v, _, _, csem = buf
        for j in range(NSUB):
            pltpu.make_async_copy(rows_v.at[pl.ds(j * SUB, SUB)],
                                  acc.at[idx_v.at[j]], csem).wait()

    def step(k, t):
        bp = bufs[t]
        bq = bufs[(t + 1) % 3]
        bn = bufs[(t + 2) % 3]

        @pl.when((k >= 2) & (k - 2 < nk))
        def _():
            wait_scatter(bq)

        @pl.when(k + 1 < nk)
        def _():
            wait_stage(bq)
            gather(bq)

        @pl.when(k < nk)
        def _():
            wait_gather(bp)
            compute(bp)
            scatter(bp)

        @pl.when(k + 2 < nk)
        def _():
            stage(k + 2, bn)

    @pl.when(0 < nk)
    def _():
        stage(0, bufs[0])

    @pl.when(1 < nk)
    def _():
        stage(1, bufs[1])

    @pl.when(0 < nk)
    def _():
        wait_stage(bufs[0])
        gather(bufs[0])

    def tri_body(m, carry):
        step(3 * m, 0)
        step(3 * m + 1, 1)
        step(3 * m + 2, 2)
        return carry

    lax.fori_loop(0, 132, tri_body, 0)

    plsc.subcore_barrier()
    pltpu.sync_copy(acc.at[pl.ds(s * ROWS_PER_TILE, ROWS_PER_TILE)],
                    out_hbm.at[pl.ds(c * HP + s * ROWS_PER_TILE, ROWS_PER_TILE)])


_spmm_call = pl.kernel(
    _spmm_body,
    out_type=jax.ShapeDtypeStruct((NP, EMB), jnp.float32),
    mesh=_mesh,
    scratch_types=(
        [pltpu.VMEM((NSUB, SUB), jnp.int32),
         pltpu.VMEM((NSUB, SUB), jnp.int32),
         pltpu.VMEM((NSUB, SUB), jnp.float32),
         pltpu.VMEM((CHUNK, EMB), jnp.float32)
         ] * 3
        + [pltpu.VMEM((64 * 16,), jnp.int32)]
        + [pltpu.VMEM_SHARED((HP, EMB), jnp.float32)]
        + [pltpu.SemaphoreType.DMA] * 9
    ),
    compiler_params=_params,
)


_BLK = 3128


def _linear_leaky_kernel(x_ref, w_ref, b_ref, o_ref):
    y = jnp.dot(x_ref[...], w_ref[...], preferred_element_type=jnp.float32)
    y = y + b_ref[...]
    o_ref[...] = jnp.where(y >= 0, y, NEG_SLOPE * y)


def _linear_leaky(x, W, b):
    return pl.pallas_call(
        _linear_leaky_kernel,
        grid=(NP // _BLK,),
        in_specs=[
            pl.BlockSpec((_BLK, EMB), lambda i: (i, 0)),
            pl.BlockSpec((EMB, EMB), lambda i: (0, 0)),
            pl.BlockSpec((1, EMB), lambda i: (0, 0)),
        ],
        out_specs=pl.BlockSpec((_BLK, EMB), lambda i: (i, 0)),
        out_shape=jax.ShapeDtypeStruct((NP, EMB), jnp.float32),
    )(x, W, b.reshape(1, EMB))


def _final_kernel(x_ref, w_ref, b_ref, e0_ref, e1_ref, o_ref):
    y = jnp.dot(x_ref[...], w_ref[...], preferred_element_type=jnp.float32)
    y = y + b_ref[...]
    y = jnp.where(y >= 0, y, NEG_SLOPE * y)
    o_ref[...] = (e0_ref[...] + e1_ref[...] + y) * (1.0 / 3.0)


def _final(side2, W, b, emb, ego1):
    return pl.pallas_call(
        _final_kernel,
        grid=(NP // _BLK,),
        in_specs=[
            pl.BlockSpec((_BLK, EMB), lambda i: (i, 0)),
            pl.BlockSpec((EMB, EMB), lambda i: (0, 0)),
            pl.BlockSpec((1, EMB), lambda i: (0, 0)),
            pl.BlockSpec((_BLK, EMB), lambda i: (i, 0)),
            pl.BlockSpec((_BLK, EMB), lambda i: (i, 0)),
        ],
        out_specs=pl.BlockSpec((_BLK, EMB), lambda i: (i, 0)),
        out_shape=jax.ShapeDtypeStruct((NP, EMB), jnp.float32),
    )(side2, W, b.reshape(1, EMB), emb, ego1)


def kernel(embedding, edge_index, edge_weight, W1, b1, W2, b2):
    emb_p = jnp.concatenate(
        [embedding[:N_HALF],
         jnp.zeros((PAD, EMB), jnp.float32),
         embedding[N_HALF:]], axis=0)
    z = jnp.zeros((ROWS_PER_TILE, EMB), jnp.float32)

    prow, pcol, pw, cnts = _part_call(
        edge_index[0], edge_index[1], edge_weight)
    side1 = _spmm_call(emb_p, prow, pcol, pw, cnts, z)
    ego1 = _linear_leaky(side1, W1, b1)
    side2 = _spmm_call(ego1, prow, pcol, pw, cnts, z)
    out = _final(side2, W2, b2, emb_p, ego1)
    return (out[:N_USERS], out[N_USERS:N_USERS + N_ITEMS])
